# Initial kernel scaffold; baseline (speedup 1.0000x reference)
import jax
import jax.numpy as jnp
from jax.experimental import pallas as pl


def kernel(token_ids, weights):
    raise NotImplementedError("write your pallas kernel here")


# speedup vs baseline: 1.8705x; 1.8705x over previous
import jax
import jax.numpy as jnp
from jax import lax
from jax.experimental import pallas as pl
from jax.experimental.pallas import tpu as pltpu
from jax.experimental.pallas import tpu_sc as plsc

NUM_EMB = 1000000
DIM = 64
BATCH = 16384
HIST = 50
B_TOTAL = BATCH * HIST
NC = 2
NS = 16
NW = NC * NS
BPW = B_TOTAL // NW
CHUNK = 128
NBUF = 8
GROUP = NBUF * CHUNK
NGROUPS = BPW // GROUP


def _emb_body(idx_hbm, table_hbm, out_hbm, idx_v, rows_v, gsem, osem):
    wid = lax.axis_index("s") * NC + lax.axis_index("c")
    base = pl.multiple_of(wid * BPW, BPW)
    pltpu.sync_copy(idx_hbm.at[pl.ds(base, BPW)], idx_v)

    @pl.loop(0, NGROUPS)
    def _group(g):
        off = pl.multiple_of(g * GROUP, GROUP)
        for b in range(NBUF):
            pltpu.async_copy(
                table_hbm.at[idx_v.at[pl.ds(off + b * CHUNK, CHUNK)]],
                rows_v.at[b],
                gsem.at[b],
            )
        for b in range(NBUF):
            pltpu.make_async_copy(
                table_hbm.at[idx_v.at[pl.ds(off + b * CHUNK, CHUNK)]],
                rows_v.at[b],
                gsem.at[b],
            ).wait()
            pltpu.async_copy(
                rows_v.at[b],
                out_hbm.at[pl.ds(base + off + b * CHUNK, CHUNK)],
                osem.at[b],
            )
        for b in range(NBUF):
            pltpu.make_async_copy(
                rows_v.at[b],
                out_hbm.at[pl.ds(base + off + b * CHUNK, CHUNK)],
                osem.at[b],
            ).wait()


@jax.jit
def _embedding_lookup(flat_ids, weights):
    mesh = plsc.VectorSubcoreMesh(core_axis_name="c", subcore_axis_name="s")
    return pl.kernel(
        _emb_body,
        out_type=jax.ShapeDtypeStruct((B_TOTAL, DIM), jnp.float32),
        mesh=mesh,
        scratch_types=[
            pltpu.VMEM((BPW,), jnp.int32),
            pltpu.VMEM((NBUF, CHUNK, DIM), jnp.float32),
            pltpu.SemaphoreType.DMA((NBUF,)),
            pltpu.SemaphoreType.DMA((NBUF,)),
        ],
        compiler_params=pltpu.CompilerParams(use_tc_tiling_on_sc=False),
    )(flat_ids, weights)


def kernel(token_ids, weights):
    flat = token_ids.reshape(-1).astype(jnp.int32)
    out = _embedding_lookup(flat, weights)
    return out.reshape(BATCH, HIST, DIM)


# speedup vs baseline: 1.8780x; 1.0040x over previous
import jax
import jax.numpy as jnp
from jax import lax
from jax.experimental import pallas as pl
from jax.experimental.pallas import tpu as pltpu
from jax.experimental.pallas import tpu_sc as plsc

NUM_EMB = 1000000
DIM = 64
BATCH = 16384
HIST = 50
B_TOTAL = BATCH * HIST
NC = 2
NS = 16
NW = NC * NS
BPW = B_TOTAL // NW
CHUNK = 128
NBUF = 8
GROUP = NBUF * CHUNK
NGROUPS = BPW // GROUP


HALF = NBUF // 2
NCHUNK = BPW // CHUNK


def _emb_body(idx_hbm, table_hbm, out_hbm, idx_v, rows_v, gsem, osem):
    wid = lax.axis_index("s") * NC + lax.axis_index("c")
    base = pl.multiple_of(wid * BPW, BPW)
    pltpu.sync_copy(idx_hbm.at[pl.ds(base, BPW)], idx_v)

    def gather_desc(i, b):
        off = pl.multiple_of(i * CHUNK, CHUNK)
        return pltpu.make_async_copy(
            table_hbm.at[idx_v.at[pl.ds(off, CHUNK)]], rows_v.at[b], gsem.at[b]
        )

    def out_desc(i, b):
        off = pl.multiple_of(i * CHUNK, CHUNK)
        return pltpu.make_async_copy(
            rows_v.at[b], out_hbm.at[pl.ds(base + off, CHUNK)], osem.at[b]
        )

    @pl.loop(0, NGROUPS)
    def _group(g):
        for b in range(NBUF):
            i = g * NBUF + b

            @pl.when(g > 0)
            def _free_slot():
                out_desc(i - NBUF, b).wait()

            gather_desc(i, b).start()

            bj = (b - HALF) % NBUF
            if b >= HALF:
                gather_desc(i - HALF, bj).wait()
                out_desc(i - HALF, bj).start()
            else:

                @pl.when(g > 0)
                def _drain_prev():
                    gather_desc(i - HALF, bj).wait()
                    out_desc(i - HALF, bj).start()

    for b in range(HALF):
        i = NCHUNK - HALF + b
        bj = i % NBUF
        gather_desc(i, bj).wait()
        out_desc(i, bj).start()
    for b in range(NBUF):
        i = NCHUNK - NBUF + b
        out_desc(i, i % NBUF).wait()


@jax.jit
def _embedding_lookup(flat_ids, weights):
    mesh = plsc.VectorSubcoreMesh(core_axis_name="c", subcore_axis_name="s")
    return pl.kernel(
        _emb_body,
        out_type=jax.ShapeDtypeStruct((B_TOTAL, DIM), jnp.float32),
        mesh=mesh,
        scratch_types=[
            pltpu.VMEM((BPW,), jnp.int32),
            pltpu.VMEM((NBUF, CHUNK, DIM), jnp.float32),
            pltpu.SemaphoreType.DMA((NBUF,)),
            pltpu.SemaphoreType.DMA((NBUF,)),
        ],
        compiler_params=pltpu.CompilerParams(use_tc_tiling_on_sc=False),
    )(flat_ids, weights)


def kernel(token_ids, weights):
    flat = token_ids.reshape(-1).astype(jnp.int32)
    out = _embedding_lookup(flat, weights)
    return out.reshape(BATCH, HIST, DIM)
